# baseline (speedup 1.0000x reference)
import functools

import jax
import jax.numpy as jnp
from jax import lax
from jax.experimental import pallas as pl
from jax.experimental.pallas import tpu as pltpu
from jax.experimental.pallas import tpu_sc as plsc

EMBED_DIM = 64
SPAD = 56


@functools.lru_cache(maxsize=None)
def _make_gather(N, S, D):
    info = plsc.get_sparse_core_info()
    NC, NS = info.num_cores, info.num_subcores
    NW = NC * NS
    assert N % NW == 0
    n_per_w = N // NW
    CHN = 16
    assert n_per_w % CHN == 0
    n_ch = n_per_w // CHN

    mesh = plsc.VectorSubcoreMesh(core_axis_name="c", subcore_axis_name="s")

    @functools.partial(
        pl.kernel,
        mesh=mesh,
        compiler_params=pltpu.CompilerParams(use_tc_tiling_on_sc=False),
        out_type=jax.ShapeDtypeStruct((N, S, D), jnp.float32),
        scratch_types=[
            pltpu.VMEM((n_per_w, SPAD), jnp.int32),
            pltpu.VMEM((CHN, SPAD, D), jnp.float32),
            pltpu.VMEM((CHN, SPAD, D), jnp.float32),
            pltpu.SemaphoreType.DMA,
            pltpu.SemaphoreType.DMA,
            pltpu.SemaphoreType.DMA,
            pltpu.SemaphoreType.DMA,
        ],
    )
    def gather_kernel(idx_hbm, table_hbm, out_hbm,
                      idx_v, rows_v0, rows_v1,
                      si, sg0, sg1, so):
        wid = lax.axis_index("s") * NC + lax.axis_index("c")
        base_n = wid * n_per_w
        rows_bufs = (rows_v0, rows_v1)
        sg = (sg0, sg1)
        pltpu.async_copy(
            idx_hbm.at[pl.ds(base_n, n_per_w)], idx_v, si).wait()
        copies_o = [[], []]
        for j in range(n_ch):
            b = j % 2
            for c in copies_o[b]:
                c.wait()
            gathers = [
                pltpu.async_copy(
                    table_hbm.at[idx_v.at[j * CHN + i]],
                    rows_bufs[b].at[i], sg[b])
                for i in range(CHN)
            ]
            for g in gathers:
                g.wait()
            copies_o[b] = [
                pltpu.async_copy(
                    rows_bufs[b].at[i, pl.ds(0, S)],
                    out_hbm.at[base_n + j * CHN + i], so)
                for i in range(CHN)
            ]
        for b in range(2):
            for c in copies_o[b]:
                c.wait()

    return gather_kernel


def kernel(idx, embeddings):
    n, s = idx.shape
    idx_p = jnp.pad(idx.astype(jnp.int32), ((0, 0), (0, SPAD - s)))
    return _make_gather(n, s, EMBED_DIM)(idx_p, embeddings)


# speedup vs baseline: 1.6791x; 1.6791x over previous
import functools

import jax
import jax.numpy as jnp
from jax import lax
from jax.experimental import pallas as pl
from jax.experimental.pallas import tpu as pltpu
from jax.experimental.pallas import tpu_sc as plsc

EMBED_DIM = 64
SPAD = 128
LANES = 16


@functools.lru_cache(maxsize=None)
def _make_gather(N, S, D):
    info = plsc.get_sparse_core_info()
    NC, NS = info.num_cores, info.num_subcores
    NW = NC * NS
    assert N % NW == 0
    n_per_w = N // NW
    b_per_w = n_per_w * S
    CH = 800
    assert b_per_w % CH == 0 and b_per_w % LANES == 0
    n_ch = b_per_w // CH

    mesh = plsc.VectorSubcoreMesh(core_axis_name="c", subcore_axis_name="s")

    @functools.partial(
        pl.kernel,
        mesh=mesh,
        compiler_params=pltpu.CompilerParams(
            use_tc_tiling_on_sc=False, needs_layout_passes=False),
        out_type=jax.ShapeDtypeStruct((N * S, D), jnp.float32),
        scratch_types=[
            pltpu.VMEM((n_per_w, SPAD), jnp.int32),
            pltpu.VMEM((b_per_w,), jnp.int32),
            pltpu.VMEM((CH, D), jnp.float32),
            pltpu.VMEM((CH, D), jnp.float32),
            pltpu.SemaphoreType.DMA,
            pltpu.SemaphoreType.DMA,
            pltpu.SemaphoreType.DMA,
            pltpu.SemaphoreType.DMA,
        ],
    )
    def gather_kernel(idx_hbm, table_hbm, out_hbm,
                      idx2d_v, idx1d_v, rows_v0, rows_v1,
                      si, sg0, sg1, so):
        wid = lax.axis_index("s") * NC + lax.axis_index("c")
        base_n = wid * n_per_w
        base = base_n * S
        rows_bufs = (rows_v0, rows_v1)
        sg = (sg0, sg1)
        pltpu.async_copy(
            idx_hbm.at[pl.ds(base_n, n_per_w)], idx2d_v, si).wait()

        lanes = lax.iota(jnp.int32, LANES)
        zeros = jnp.zeros((LANES,), jnp.int32)

        def body(m, carry):
            rows, cols = carry
            vals = plsc.load_gather(idx2d_v, [rows, cols])
            idx1d_v[pl.ds(pl.multiple_of(m * LANES, LANES), LANES)] = vals
            cols2 = cols + LANES
            wrap = cols2 >= S
            rows2 = jnp.where(wrap, rows + 1, rows)
            cols3 = jnp.where(wrap, cols2 - S, cols2)
            return rows2, cols3

        lax.fori_loop(0, b_per_w // LANES, body, (zeros, lanes))

        copies_o = [None, None]
        for j in range(n_ch):
            b = j % 2
            if copies_o[b] is not None:
                copies_o[b].wait()
            gather = pltpu.async_copy(
                table_hbm.at[idx1d_v.at[pl.ds(j * CH, CH)]],
                rows_bufs[b], sg[b])
            gather.wait()
            copies_o[b] = pltpu.async_copy(
                rows_bufs[b], out_hbm.at[pl.ds(base + j * CH, CH)], so)
        if copies_o[(n_ch - 2) % 2] is not None:
            copies_o[(n_ch - 2) % 2].wait()
        copies_o[(n_ch - 1) % 2].wait()

    return gather_kernel


def kernel(idx, embeddings):
    n, s = idx.shape
    idx_p = jnp.pad(idx.astype(jnp.int32), ((0, 0), (0, SPAD - s)))
    out = _make_gather(n, s, EMBED_DIM)(idx_p, embeddings)
    return out.reshape(n, s, EMBED_DIM)
